# baseline (speedup 1.0000x reference)
import functools

import jax
import jax.numpy as jnp
from jax import lax
from jax.experimental import pallas as pl
from jax.experimental.pallas import tpu as pltpu
from jax.experimental.pallas import tpu_sc as plsc

_INFO = plsc.get_sparse_core_info()
_NC = _INFO.num_cores
_NS = _INFO.num_subcores
_NW = _NC * _NS
_L = 16
_CHUNK = 4096


def _build(B, V, D):
    feats_per_w = D // _NW
    n_chunks = B // _CHUNK
    mesh = plsc.VectorSubcoreMesh(core_axis_name="c", subcore_axis_name="s")

    @functools.partial(
        pl.kernel,
        mesh=mesh,
        out_type=jax.ShapeDtypeStruct((D, B), jnp.float32),
        compiler_params=pltpu.CompilerParams(
            use_tc_tiling_on_sc=True,
            needs_layout_passes=False,
            disable_bounds_checks=True,
            disable_semaphore_checks=True,
            skip_device_barrier=True,
        ),
        scratch_types=[
            pltpu.VMEM((V,), jnp.float32),
            pltpu.VMEM((B,), jnp.int32),
            pltpu.VMEM((2, _CHUNK), jnp.float32),
            pltpu.SemaphoreType.DMA,
            pltpu.SemaphoreType.DMA,
        ],
    )
    def k(table_t_hbm, labels_hbm, out_t_hbm, fv, idx_v, st, sem_f, sem_o):
        wid = lax.axis_index("s") * _NC + lax.axis_index("c")

        def fv_copy(f):
            c = wid * feats_per_w + f
            return pltpu.make_async_copy(table_t_hbm.at[c], fv, sem_f)

        def store_copy(f, ch, buf):
            c = wid * feats_per_w + f
            return pltpu.make_async_copy(
                st.at[buf], out_t_hbm.at[c, pl.ds(ch * _CHUNK, _CHUNK)], sem_o
            )

        with jax.named_scope("ph_fv0"):
            fv_copy(0).start()
            pltpu.sync_copy(labels_hbm, idx_v)
            fv_copy(0).wait()

        for f in range(feats_per_w):
            if f > 0:
                with jax.named_scope(f"ph_fv{f}"):
                    fv_copy(f).start()
                    fv_copy(f).wait()
            for ch in range(n_chunks):
                g = f * n_chunks + ch
                buf = g & 1
                if g >= 2:
                    prev = g - 2
                    store_copy(prev // n_chunks, prev % n_chunks, buf).wait()

                with jax.named_scope(f"ph_g{f}_{ch}"):

                    @plsc.parallel_loop(0, _CHUNK // _L, unroll=8)
                    def gather(j):
                        ii = idx_v[pl.ds(ch * _CHUNK + j * _L, _L)]
                        st[buf, pl.ds(j * _L, _L)] = plsc.load_gather(fv, [ii])

                    store_copy(f, ch, buf).start()

        total = feats_per_w * n_chunks
        for g in (total - 2, total - 1):
            store_copy(g // n_chunks, g % n_chunks, g & 1).wait()

    return k


def kernel(labels, embedding_table):
    B = labels.shape[0]
    V, D = embedding_table.shape
    out_t = _build(B, V, D)(embedding_table.T, labels.astype(jnp.int32))
    return out_t.T


# speedup vs baseline: 1.0061x; 1.0061x over previous
import functools

import jax
import jax.numpy as jnp
from jax import lax
from jax.experimental import pallas as pl
from jax.experimental.pallas import tpu as pltpu
from jax.experimental.pallas import tpu_sc as plsc

_INFO = plsc.get_sparse_core_info()
_NC = _INFO.num_cores
_NS = _INFO.num_subcores
_NW = _NC * _NS
_L = 16
_CHUNK = 4096


def _build(B, V, D):
    feats_per_w = D // _NW
    n_chunks = B // _CHUNK
    mesh = plsc.VectorSubcoreMesh(core_axis_name="c", subcore_axis_name="s")

    @functools.partial(
        pl.kernel,
        mesh=mesh,
        out_type=jax.ShapeDtypeStruct((D, B), jnp.float32),
        compiler_params=pltpu.CompilerParams(
            use_tc_tiling_on_sc=True,
            needs_layout_passes=False,
        ),
        scratch_types=[
            pltpu.VMEM((V,), jnp.float32),
            pltpu.VMEM((B,), jnp.int32),
            pltpu.VMEM((2, _CHUNK), jnp.float32),
            pltpu.SemaphoreType.DMA,
            pltpu.SemaphoreType.DMA,
        ],
    )
    def k(table_t_hbm, labels_hbm, out_t_hbm, fv, idx_v, st, sem_f, sem_o):
        wid = lax.axis_index("s") * _NC + lax.axis_index("c")

        def fv_copy(f):
            c = wid * feats_per_w + f
            return pltpu.make_async_copy(table_t_hbm.at[c], fv, sem_f)

        def store_copy(f, ch, buf):
            c = wid * feats_per_w + f
            return pltpu.make_async_copy(
                st.at[buf], out_t_hbm.at[c, pl.ds(ch * _CHUNK, _CHUNK)], sem_o
            )

        fv_copy(0).start()
        pltpu.sync_copy(labels_hbm, idx_v)
        fv_copy(0).wait()

        for f in range(feats_per_w):
            if f > 0:
                fv_copy(f).start()
                fv_copy(f).wait()
            for ch in range(n_chunks):
                g = f * n_chunks + ch
                buf = g & 1
                if g >= 2:
                    prev = g - 2
                    store_copy(prev // n_chunks, prev % n_chunks, buf).wait()

                @plsc.parallel_loop(0, _CHUNK // _L, unroll=8)
                def gather(j):
                    ii = idx_v[pl.ds(ch * _CHUNK + j * _L, _L)]
                    st[buf, pl.ds(j * _L, _L)] = plsc.load_gather(fv, [ii])

                store_copy(f, ch, buf).start()

        total = feats_per_w * n_chunks
        for g in (total - 2, total - 1):
            store_copy(g // n_chunks, g % n_chunks, g & 1).wait()

    return k


def kernel(labels, embedding_table):
    B = labels.shape[0]
    V, D = embedding_table.shape
    out_t = _build(B, V, D)(embedding_table.T, labels.astype(jnp.int32))
    return out_t.T


# speedup vs baseline: 1.1075x; 1.1008x over previous
import functools

import jax
import jax.numpy as jnp
from jax import lax
from jax.experimental import pallas as pl
from jax.experimental.pallas import tpu as pltpu
from jax.experimental.pallas import tpu_sc as plsc

_INFO = plsc.get_sparse_core_info()
_NC = _INFO.num_cores
_NS = _INFO.num_subcores
_NW = _NC * _NS
_L = 16
_CHUNK = 4096


def _build(B, V, D):
    feats_per_w = D // _NW
    n_chunks = B // _CHUNK
    mesh = plsc.VectorSubcoreMesh(core_axis_name="c", subcore_axis_name="s")

    @functools.partial(
        pl.kernel,
        mesh=mesh,
        out_type=jax.ShapeDtypeStruct((D, B), jnp.float32),
        compiler_params=pltpu.CompilerParams(
            use_tc_tiling_on_sc=True,
            needs_layout_passes=False,
        ),
        scratch_types=[
            pltpu.VMEM((V,), jnp.float32),
            pltpu.VMEM((B,), jnp.int32),
            pltpu.VMEM((2, _CHUNK), jnp.float32),
            pltpu.VMEM_SHARED((B,), jnp.int32),
            pltpu.SemaphoreType.DMA,
            pltpu.SemaphoreType.DMA,
        ],
    )
    def k(table_t_hbm, labels_hbm, out_t_hbm, fv, idx_v, st, idx_sh, sem_f, sem_o):
        wid = lax.axis_index("s") * _NC + lax.axis_index("c")

        def fv_copy(f):
            c = wid * feats_per_w + f
            return pltpu.make_async_copy(table_t_hbm.at[c], fv, sem_f)

        def store_copy(f, ch, buf):
            c = wid * feats_per_w + f
            return pltpu.make_async_copy(
                st.at[buf], out_t_hbm.at[c, pl.ds(ch * _CHUNK, _CHUNK)], sem_o
            )

        fv_copy(0).start()
        @pl.when(lax.axis_index("s") == 0)
        def _():
            pltpu.sync_copy(labels_hbm, idx_sh)

        plsc.subcore_barrier()
        pltpu.sync_copy(idx_sh, idx_v)
        fv_copy(0).wait()

        for f in range(feats_per_w):
            if f > 0:
                fv_copy(f).start()
                fv_copy(f).wait()
            for ch in range(n_chunks):
                g = f * n_chunks + ch
                buf = g & 1
                if g >= 2:
                    prev = g - 2
                    store_copy(prev // n_chunks, prev % n_chunks, buf).wait()

                @plsc.parallel_loop(0, _CHUNK // _L, unroll=8)
                def gather(j):
                    ii = idx_v[pl.ds(ch * _CHUNK + j * _L, _L)]
                    st[buf, pl.ds(j * _L, _L)] = plsc.load_gather(fv, [ii])

                store_copy(f, ch, buf).start()

        total = feats_per_w * n_chunks
        for g in (total - 2, total - 1):
            store_copy(g // n_chunks, g % n_chunks, g & 1).wait()

    return k


def kernel(labels, embedding_table):
    B = labels.shape[0]
    V, D = embedding_table.shape
    out_t = _build(B, V, D)(embedding_table.T, labels.astype(jnp.int32))
    return out_t.T
